# baseline (speedup 1.0000x reference)
import functools

import jax
import jax.numpy as jnp
from jax import lax
from jax.experimental import pallas as pl
from jax.experimental.pallas import tpu as pltpu
from jax.experimental.pallas import tpu_sc as plsc

N = 10000
NP = 10240
E = 160000
TILES = 16
EBP = 80
NB = E // (TILES * EBP)
RT = NP // TILES

_MESH = plsc.VectorSubcoreMesh(core_axis_name="c", subcore_axis_name="s")
_F32 = jnp.float32


def _deg_body(dst_hbm, ones_hbm, zeros_hbm, o0_hbm, o1_hbm,
              dst_v, ones_v, acc, sem):
    cc = lax.axis_index("c")
    s = lax.axis_index("s")
    pltpu.sync_copy(dst_hbm.at[s], dst_v)
    pltpu.sync_copy(ones_hbm, ones_v)
    pltpu.sync_copy(zeros_hbm, acc.at[pl.ds(s * RT, RT)])
    plsc.subcore_barrier()
    lo = cc * (NB // 2)
    hi = lo + NB // 2 + cc * (NB % 2)

    def it(j, carry):
        pltpu.async_copy(ones_v, acc.at[dst_v.at[j]], sem, add=True)

        @pl.when(j >= lo + 8)
        def _():
            pltpu.make_async_copy(ones_v, acc.at[dst_v.at[j]], sem).wait()

        return carry

    lax.fori_loop(lo, hi, it, 0)
    for _ in range(8):
        pltpu.make_async_copy(ones_v, acc.at[dst_v.at[0]], sem).wait()
    plsc.subcore_barrier()
    for core_id in range(2):
        out = (o0_hbm, o1_hbm)[core_id]

        @pl.when(cc == core_id)
        def _(out=out):
            pltpu.sync_copy(acc.at[pl.ds(s * RT, RT)],
                            out.at[pl.ds(s * RT, RT)])


_deg_kernel = functools.partial(
    pl.kernel,
    out_type=[jax.ShapeDtypeStruct((NP, 128), _F32)] * 2,
    mesh=_MESH,
    scratch_types=[
        pltpu.VMEM((NB, EBP), jnp.int32),
        pltpu.VMEM((EBP, 128), _F32),
        pltpu.VMEM_SHARED((NP, 128), _F32),
        pltpu.SemaphoreType.DMA,
    ],
)(_deg_body)


def _make_agg_body(C):
    C2 = C // 2

    def body(src_hbm, dst_hbm, *rest):
        tables = rest[:C]
        zeros_hbm = rest[C]
        outs = rest[C + 1:2 * C + 1]
        src_v, ring, rows, acc = rest[2 * C + 1:2 * C + 5]
        gsems = rest[2 * C + 5:2 * C + 7]
        ssems = rest[2 * C + 7:2 * C + 9]
        isems = rest[2 * C + 9:2 * C + 13]

        cc = lax.axis_index("c")
        s = lax.axis_index("s")
        pltpu.sync_copy(src_hbm.at[s], src_v)

        def run_chunk(table, out):
            pltpu.sync_copy(zeros_hbm, acc.at[pl.ds(s * RT, RT)])
            plsc.subcore_barrier()
            for r in range(2):
                pltpu.async_copy(dst_hbm.at[s, r], ring.at[r], isems[r])

            def group(g, carry):
                j0 = g * 4
                for t in range(4):
                    j = j0 + t
                    b = t % 2

                    @pl.when(j >= 2)
                    def _(b=b, t=t):
                        pltpu.make_async_copy(
                            rows.at[b], acc.at[ring.at[t, 0]], ssems[b]).wait()

                    jp = j + 2
                    rp = (t + 2) % 4

                    @pl.when(jp < NB)
                    def _(jp=jp, rp=rp):
                        pltpu.async_copy(
                            dst_hbm.at[s, jp], ring.at[rp], isems[rp])

                    pltpu.async_copy(
                        table.at[src_v.at[j]], rows.at[b], gsems[b]).wait()
                    pltpu.make_async_copy(
                        dst_hbm.at[s, j], ring.at[t], isems[t]).wait()
                    pltpu.async_copy(
                        rows.at[b], acc.at[ring.at[t, 0]], ssems[b], add=True)
                return carry

            lax.fori_loop(0, NB // 4, group, 0)
            j_last = NB - 1
            pltpu.make_async_copy(
                rows.at[0], acc.at[ring.at[0, 0]], ssems[0]).wait()
            pltpu.async_copy(
                table.at[src_v.at[j_last]], rows.at[0], gsems[0]).wait()
            pltpu.make_async_copy(
                dst_hbm.at[s, j_last], ring.at[0], isems[0]).wait()
            pltpu.async_copy(
                rows.at[0], acc.at[ring.at[0, 0]], ssems[0], add=True)
            for b in range(2):
                pltpu.make_async_copy(
                    rows.at[b], acc.at[ring.at[b, 0]], ssems[b]).wait()
            plsc.subcore_barrier()
            pltpu.sync_copy(acc.at[pl.ds(s * RT, RT)],
                            out.at[pl.ds(s * RT, RT)])
            plsc.subcore_barrier()

        for core_id in range(2):
            @pl.when(cc == core_id)
            def _(core_id=core_id):
                for k in range(C2):
                    ch = core_id * C2 + k
                    run_chunk(tables[ch], outs[ch])

    return body


def _make_agg(C):
    return functools.partial(
        pl.kernel,
        out_type=[jax.ShapeDtypeStruct((NP, 128), _F32) for _ in range(C)],
        mesh=_MESH,
        scratch_types=[
            pltpu.VMEM((NB, EBP), jnp.int32),
            pltpu.VMEM((4, 1, EBP), jnp.int32),
            pltpu.VMEM((2, EBP, 128), _F32),
            pltpu.VMEM_SHARED((NP, 128), _F32),
        ] + [pltpu.SemaphoreType.DMA] * 8,
    )(_make_agg_body(C))


_agg2 = _make_agg(2)
_agg4 = _make_agg(4)


_BN = 1000


def _prep_body(deg_ref, x_ref, xp0_ref, xp1_ref):
    dinv = lax.rsqrt(deg_ref[:, 0:1] + 1.0)
    xp = x_ref[...] * dinv
    xp0_ref[...] = xp[:, :128]
    xp1_ref[...] = xp[:, 128:]


def _mid_body(deg_ref, s10, s11, xp0, xp1, w1, b1, w2, gp0, gp1, gp2, gp3):
    dinv = lax.rsqrt(deg_ref[:, 0:1] + 1.0)
    u1 = jnp.concatenate(
        [s10[...] + xp0[...], s11[...] + xp1[...]], axis=1) * dinv
    h1 = jnp.maximum(
        jnp.dot(u1, w1[...], preferred_element_type=_F32) + b1[...], 0.0)
    g = jnp.dot(h1, w2[...], preferred_element_type=_F32) * dinv
    gp0[...] = g[:, 0:128]
    gp1[...] = g[:, 128:256]
    gp2[...] = g[:, 256:384]
    gp3[...] = g[:, 384:512]


def _out_body(deg_ref, s20, s21, s22, s23, gp0, gp1, gp2, gp3, b2, w3, b3,
              out_ref):
    dinv = lax.rsqrt(deg_ref[:, 0:1] + 1.0)
    u2 = jnp.concatenate(
        [s20[...] + gp0[...], s21[...] + gp1[...],
         s22[...] + gp2[...], s23[...] + gp3[...]], axis=1) * dinv + b2[...]
    h2 = jnp.maximum(u2, 0.0)
    logits = jnp.dot(h2, w3[...], preferred_element_type=_F32) + b3[...]
    m = jnp.max(logits, axis=1, keepdims=True)
    p = jnp.exp(logits - m)
    out_ref[...] = p / jnp.sum(p, axis=1, keepdims=True)


def _row_spec(w):
    return pl.BlockSpec((_BN, w), lambda n: (n, 0))


def _full_spec(shape):
    return pl.BlockSpec(shape, lambda n: tuple(0 for _ in shape))


_prep = pl.pallas_call(
    _prep_body,
    grid=(N // _BN,),
    in_specs=[_row_spec(16), _row_spec(256)],
    out_specs=[_row_spec(128), _row_spec(128)],
    out_shape=[jax.ShapeDtypeStruct((N, 128), _F32)] * 2,
)

_mid = pl.pallas_call(
    _mid_body,
    grid=(N // _BN,),
    in_specs=[_row_spec(16)] + [_row_spec(128)] * 4 + [
        _full_spec((256, 512)), _full_spec((1, 512)), _full_spec((512, 512))],
    out_specs=[_row_spec(128)] * 4,
    out_shape=[jax.ShapeDtypeStruct((N, 128), _F32)] * 4,
)

_outk = pl.pallas_call(
    _out_body,
    grid=(N // _BN,),
    in_specs=[_row_spec(16)] + [_row_spec(128)] * 8 + [
        _full_spec((1, 512)), _full_spec((512, 128)), _full_spec((1, 128))],
    out_specs=_row_spec(128),
    out_shape=jax.ShapeDtypeStruct((N, 128), _F32),
)


def kernel(x, edge_index, W1, b1, W2, b2, W3, b3):
    src = edge_index[0].reshape(TILES, NB, EBP)
    dst = edge_index[1].reshape(TILES, NB, EBP)
    ones128 = jnp.ones((EBP, 128), _F32)
    zeros128 = jnp.zeros((RT, 128), _F32)

    d0, d1 = _deg_kernel(dst, ones128, zeros128)
    deg16 = d0[:N, :16] + d1[:N, :16]
    xp0, xp1 = _prep(deg16, x)
    dst4 = dst.reshape(TILES, NB, 1, EBP)
    s1 = _agg2(src, dst4, xp0, xp1, zeros128)
    gps = _mid(deg16, s1[0][:N], s1[1][:N], xp0, xp1,
               W1, b1.reshape(1, -1), W2)
    s2 = _agg4(src, dst4, *gps, zeros128)
    return _outk(deg16, *(s[:N] for s in s2), *gps,
                 b2.reshape(1, -1), W3, b3.reshape(1, -1))


# speedup vs baseline: 1.3708x; 1.3708x over previous
import functools

import jax
import jax.numpy as jnp
from jax import lax
from jax.experimental import pallas as pl
from jax.experimental.pallas import tpu as pltpu
from jax.experimental.pallas import tpu_sc as plsc

N = 10000
NP = 10240
E = 160000
TILES = 16
EBP = 80
NB = E // (TILES * EBP)
RT = NP // TILES

_MESH = plsc.VectorSubcoreMesh(core_axis_name="c", subcore_axis_name="s")
_F32 = jnp.float32


def _deg_body(dst_hbm, ones_hbm, zeros_hbm, o0_hbm, o1_hbm,
              dst_v, ones_v, acc, sem):
    cc = lax.axis_index("c")
    s = lax.axis_index("s")
    pltpu.sync_copy(dst_hbm.at[s], dst_v)
    pltpu.sync_copy(ones_hbm, ones_v)
    pltpu.sync_copy(zeros_hbm, acc.at[pl.ds(s * RT, RT)])
    plsc.subcore_barrier()
    lo = cc * (NB // 2)
    hi = lo + NB // 2 + cc * (NB % 2)

    def it(j, carry):
        pltpu.async_copy(ones_v, acc.at[dst_v.at[j]], sem, add=True)

        @pl.when(j >= lo + 8)
        def _():
            pltpu.make_async_copy(ones_v, acc.at[dst_v.at[j]], sem).wait()

        return carry

    lax.fori_loop(lo, hi, it, 0)
    for _ in range(8):
        pltpu.make_async_copy(ones_v, acc.at[dst_v.at[0]], sem).wait()
    plsc.subcore_barrier()
    for core_id in range(2):
        out = (o0_hbm, o1_hbm)[core_id]

        @pl.when(cc == core_id)
        def _(out=out):
            pltpu.sync_copy(acc.at[pl.ds(s * RT, RT)],
                            out.at[pl.ds(s * RT, RT)])


_deg_kernel = functools.partial(
    pl.kernel,
    out_type=[jax.ShapeDtypeStruct((NP, 128), _F32)] * 2,
    mesh=_MESH,
    scratch_types=[
        pltpu.VMEM((NB, EBP), jnp.int32),
        pltpu.VMEM((EBP, 128), _F32),
        pltpu.VMEM_SHARED((NP, 128), _F32),
        pltpu.SemaphoreType.DMA,
    ],
)(_deg_body)


def _make_agg_body(C):
    C2 = C // 2

    def body(src_hbm, dst_hbm, *rest):
        tables = rest[:C]
        zeros_hbm = rest[C]
        outs = rest[C + 1:2 * C + 1]
        src_v, ring, rows, acc = rest[2 * C + 1:2 * C + 5]
        gsems = rest[2 * C + 5:2 * C + 8]
        ssems = rest[2 * C + 8:2 * C + 11]
        isems = rest[2 * C + 11:2 * C + 17]

        cc = lax.axis_index("c")
        s = lax.axis_index("s")
        pltpu.sync_copy(src_hbm.at[s], src_v)

        def run_chunk(table, out):
            pltpu.sync_copy(zeros_hbm, acc.at[pl.ds(s * RT, RT)])
            plsc.subcore_barrier()
            for r in range(5):
                pltpu.async_copy(dst_hbm.at[s, r], ring.at[r], isems[r])
            for q in range(2):
                pltpu.async_copy(table.at[src_v.at[q]], rows.at[q], gsems[q])

            def step(j, jj):
                b = j % 3
                r = j % 6
                pltpu.make_async_copy(
                    table.at[src_v.at[jj]], rows.at[b], gsems[b]).wait()
                pltpu.make_async_copy(
                    dst_hbm.at[s, jj], ring.at[r], isems[r]).wait()
                pltpu.async_copy(
                    rows.at[b], acc.at[ring.at[r, 0]], ssems[b], add=True)
                if j >= 1:
                    bp = (j - 1) % 3
                    pltpu.make_async_copy(
                        rows.at[bp], acc.at[ring.at[r, 0]], ssems[bp]).wait()
                if j + 2 < NB:
                    bg = (j + 2) % 3
                    pltpu.async_copy(
                        table.at[src_v.at[jj + 2]], rows.at[bg], gsems[bg])
                if j + 5 < NB:
                    rl = (j + 5) % 6
                    pltpu.async_copy(
                        dst_hbm.at[s, jj + 5], ring.at[rl], isems[rl])

            for j in range(6):
                step(j, j)

            def group(g, carry):
                j0 = g * 6 + 6
                for t in range(6):
                    step(6 + t, j0 + t)
                return carry

            n_tail = (NB - 6) % 6
            lax.fori_loop(0, (NB - 6) // 6, group, 0)
            for j in range(NB - n_tail, NB):
                step(j, j)
            pltpu.make_async_copy(
                rows.at[(NB - 1) % 3], acc.at[ring.at[0, 0]],
                ssems[(NB - 1) % 3]).wait()
            plsc.subcore_barrier()
            pltpu.sync_copy(acc.at[pl.ds(s * RT, RT)],
                            out.at[pl.ds(s * RT, RT)])
            plsc.subcore_barrier()

        for core_id in range(2):
            @pl.when(cc == core_id)
            def _(core_id=core_id):
                for k in range(C2):
                    ch = core_id * C2 + k
                    run_chunk(tables[ch], outs[ch])

    return body


def _make_agg(C):
    return functools.partial(
        pl.kernel,
        out_type=[jax.ShapeDtypeStruct((NP, 128), _F32) for _ in range(C)],
        mesh=_MESH,
        scratch_types=[
            pltpu.VMEM((NB, EBP), jnp.int32),
            pltpu.VMEM((6, 1, EBP), jnp.int32),
            pltpu.VMEM((3, EBP, 128), _F32),
            pltpu.VMEM_SHARED((NP, 128), _F32),
        ] + [pltpu.SemaphoreType.DMA] * 12,
    )(_make_agg_body(C))


_agg2 = _make_agg(2)
_agg4 = _make_agg(4)


_BN = 1000


def _prep_body(deg_ref, x_ref, xp0_ref, xp1_ref):
    dinv = lax.rsqrt(deg_ref[:, 0:1] + 1.0)
    xp = x_ref[...] * dinv
    xp0_ref[...] = xp[:, :128]
    xp1_ref[...] = xp[:, 128:]


def _mid_body(deg_ref, s10, s11, xp0, xp1, w1, b1, w2, gp0, gp1, gp2, gp3):
    dinv = lax.rsqrt(deg_ref[:, 0:1] + 1.0)
    u1 = jnp.concatenate(
        [s10[...] + xp0[...], s11[...] + xp1[...]], axis=1) * dinv
    h1 = jnp.maximum(
        jnp.dot(u1, w1[...], preferred_element_type=_F32) + b1[...], 0.0)
    g = jnp.dot(h1, w2[...], preferred_element_type=_F32) * dinv
    gp0[...] = g[:, 0:128]
    gp1[...] = g[:, 128:256]
    gp2[...] = g[:, 256:384]
    gp3[...] = g[:, 384:512]


def _out_body(deg_ref, s20, s21, s22, s23, gp0, gp1, gp2, gp3, b2, w3, b3,
              out_ref):
    dinv = lax.rsqrt(deg_ref[:, 0:1] + 1.0)
    u2 = jnp.concatenate(
        [s20[...] + gp0[...], s21[...] + gp1[...],
         s22[...] + gp2[...], s23[...] + gp3[...]], axis=1) * dinv + b2[...]
    h2 = jnp.maximum(u2, 0.0)
    logits = jnp.dot(h2, w3[...], preferred_element_type=_F32) + b3[...]
    m = jnp.max(logits, axis=1, keepdims=True)
    p = jnp.exp(logits - m)
    out_ref[...] = p / jnp.sum(p, axis=1, keepdims=True)


def _row_spec(w):
    return pl.BlockSpec((_BN, w), lambda n: (n, 0))


def _full_spec(shape):
    return pl.BlockSpec(shape, lambda n: tuple(0 for _ in shape))


_prep = pl.pallas_call(
    _prep_body,
    grid=(N // _BN,),
    in_specs=[_row_spec(16), _row_spec(256)],
    out_specs=[_row_spec(128), _row_spec(128)],
    out_shape=[jax.ShapeDtypeStruct((N, 128), _F32)] * 2,
)

_mid = pl.pallas_call(
    _mid_body,
    grid=(N // _BN,),
    in_specs=[_row_spec(16)] + [_row_spec(128)] * 4 + [
        _full_spec((256, 512)), _full_spec((1, 512)), _full_spec((512, 512))],
    out_specs=[_row_spec(128)] * 4,
    out_shape=[jax.ShapeDtypeStruct((N, 128), _F32)] * 4,
)

_outk = pl.pallas_call(
    _out_body,
    grid=(N // _BN,),
    in_specs=[_row_spec(16)] + [_row_spec(128)] * 8 + [
        _full_spec((1, 512)), _full_spec((512, 128)), _full_spec((1, 128))],
    out_specs=_row_spec(128),
    out_shape=jax.ShapeDtypeStruct((N, 128), _F32),
)


def kernel(x, edge_index, W1, b1, W2, b2, W3, b3):
    src = edge_index[0].reshape(TILES, NB, EBP)
    dst = edge_index[1].reshape(TILES, NB, EBP)
    ones128 = jnp.ones((EBP, 128), _F32)
    zeros128 = jnp.zeros((RT, 128), _F32)

    d0, d1 = _deg_kernel(dst, ones128, zeros128)
    deg16 = d0[:N, :16] + d1[:N, :16]
    xp0, xp1 = _prep(deg16, x)
    dst4 = dst.reshape(TILES, NB, 1, EBP)
    s1 = _agg2(src, dst4, xp0, xp1, zeros128)
    gps = _mid(deg16, s1[0][:N], s1[1][:N], xp0, xp1,
               W1, b1.reshape(1, -1), W2)
    s2 = _agg4(src, dst4, *gps, zeros128)
    return _outk(deg16, *(s[:N] for s in s2), *gps,
                 b2.reshape(1, -1), W3, b3.reshape(1, -1))
